# baseline (speedup 1.0000x reference)
import functools

import jax
import jax.numpy as jnp
from jax import lax
from jax.experimental import pallas as pl
from jax.experimental.pallas import tpu as pltpu
from jax.experimental.pallas import tpu_sc as plsc

_B = 8
_SEG = 1024
_S = 32
_L = 16
_NW = 32
_PTS = (_B * _SEG) // _NW
_WPB = _SEG // _PTS


def _sc_matched_targets(inst2d, seed_ids):
    mesh = plsc.VectorSubcoreMesh(
        core_axis_name="c", subcore_axis_name="s", num_cores=2, num_subcores=16
    )

    @functools.partial(
        pl.kernel,
        out_type=jax.ShapeDtypeStruct((_B, _SEG, _S), jnp.float32),
        mesh=mesh,
        scratch_types=[
            pltpu.VMEM((_SEG,), jnp.int32),
            pltpu.VMEM((_S,), jnp.int32),
            pltpu.VMEM((_PTS, _S), jnp.float32),
        ],
        compiler_params=pltpu.CompilerParams(needs_layout_passes=False),
    )
    def k(inst_hbm, sid_hbm, out_hbm, seg_v, sid_v, out_v):
        wid = lax.axis_index("s") * 2 + lax.axis_index("c")
        b = wid // _WPB
        chunk = wid % _WPB
        base = chunk * _PTS
        pltpu.sync_copy(inst_hbm.at[b], seg_v)
        pltpu.sync_copy(sid_hbm.at[b], sid_v)
        scls = []
        for h in range(_S // _L):
            sidx = sid_v[pl.ds(h * _L, _L)]
            scls.append(plsc.load_gather(seg_v, [sidx]))

        ones = jnp.full((_L,), 1.0, jnp.float32)
        zeros = jnp.full((_L,), 0.0, jnp.float32)

        def body(i, carry):
            pidx = base + i
            idxv = jnp.full((_L,), pidx, dtype=jnp.int32)
            pcls = plsc.load_gather(seg_v, [idxv])
            for h in range(_S // _L):
                eq = pcls == scls[h]
                out_v[i, pl.ds(h * _L, _L)] = jnp.where(eq, ones, zeros)
            return carry

        lax.fori_loop(0, _PTS, body, 0)
        pltpu.sync_copy(out_v, out_hbm.at[b, pl.ds(base, _PTS)])

    return k(inst2d, seed_ids)


def kernel(outputs_masks, instance, seed_ids, offset):
    nb = offset.shape[0]
    seg = outputs_masks.shape[0] // nb
    matched_outputs = outputs_masks.reshape(nb, seg, outputs_masks.shape[1])
    inst2d = instance.reshape(nb, seg)
    matched_targets = _sc_matched_targets(inst2d, seed_ids)
    return (matched_outputs, matched_targets)


# speedup vs baseline: 1.0210x; 1.0210x over previous
import functools

import jax
import jax.numpy as jnp
from jax import lax
from jax.experimental import pallas as pl
from jax.experimental.pallas import tpu as pltpu
from jax.experimental.pallas import tpu_sc as plsc

_B = 8
_SEG = 1024
_S = 32
_L = 16
_NW = 32
_PTS = (_B * _SEG) // _NW
_WPB = _SEG // _PTS


def _sc_matched_targets(instance, seed_ids):
    mesh = plsc.VectorSubcoreMesh(
        core_axis_name="c", subcore_axis_name="s", num_cores=2, num_subcores=16
    )

    @functools.partial(
        pl.kernel,
        out_type=jax.ShapeDtypeStruct((_B, _SEG, _S), jnp.float32),
        mesh=mesh,
        scratch_types=[
            pltpu.VMEM((_SEG,), jnp.int32),
            pltpu.VMEM((_S,), jnp.int32),
            pltpu.VMEM((_PTS, _S), jnp.float32),
        ],
        compiler_params=pltpu.CompilerParams(needs_layout_passes=False),
    )
    def k(inst_hbm, sid_hbm, out_hbm, seg_v, sid_v, out_v):
        wid = lax.axis_index("s") * 2 + lax.axis_index("c")
        b = wid // _WPB
        chunk = wid % _WPB
        base = chunk * _PTS
        pltpu.sync_copy(inst_hbm.at[pl.ds(b * _SEG, _SEG)], seg_v)
        pltpu.sync_copy(sid_hbm.at[b], sid_v)
        scls = []
        for h in range(_S // _L):
            sidx = sid_v[pl.ds(h * _L, _L)]
            scls.append(plsc.load_gather(seg_v, [sidx]))

        ones = jnp.full((_L,), 1.0, jnp.float32)
        zeros = jnp.full((_L,), 0.0, jnp.float32)

        def body(i, carry):
            pidx = base + i
            idxv = jnp.full((_L,), pidx, dtype=jnp.int32)
            pcls = plsc.load_gather(seg_v, [idxv])
            for h in range(_S // _L):
                eq = pcls == scls[h]
                out_v[i, pl.ds(h * _L, _L)] = jnp.where(eq, ones, zeros)
            return carry

        lax.fori_loop(0, _PTS, body, 0)
        pltpu.sync_copy(out_v, out_hbm.at[b, pl.ds(base, _PTS)])

    return k(instance, seed_ids)


def kernel(outputs_masks, instance, seed_ids, offset):
    nb = offset.shape[0]
    seg = outputs_masks.shape[0] // nb
    matched_outputs = outputs_masks.reshape(nb, seg, outputs_masks.shape[1])
    matched_targets = _sc_matched_targets(instance, seed_ids)
    return (matched_outputs, matched_targets)


# speedup vs baseline: 1.0364x; 1.0151x over previous
import functools

import jax
import jax.numpy as jnp
from jax import lax
from jax.experimental import pallas as pl
from jax.experimental.pallas import tpu as pltpu
from jax.experimental.pallas import tpu_sc as plsc

_B = 8
_SEG = 1024
_S = 32
_L = 16
_NW = 32
_PTS = (_B * _SEG) // _NW
_WPB = _SEG // _PTS


def _sc_matched_targets(instance, seed_ids):
    mesh = plsc.VectorSubcoreMesh(
        core_axis_name="c", subcore_axis_name="s", num_cores=2, num_subcores=16
    )

    @functools.partial(
        pl.kernel,
        out_type=jax.ShapeDtypeStruct((_B, _SEG, _S), jnp.float32),
        mesh=mesh,
        scratch_types=[
            pltpu.VMEM((_SEG,), jnp.int32),
            pltpu.VMEM((_S,), jnp.int32),
            pltpu.VMEM((_PTS, _S), jnp.float32),
        ],
        compiler_params=pltpu.CompilerParams(needs_layout_passes=False),
    )
    def k(inst_hbm, sid_hbm, out_hbm, seg_v, sid_v, out_v):
        wid = lax.axis_index("s") * 2 + lax.axis_index("c")
        b = wid // _WPB
        chunk = wid % _WPB
        base = chunk * _PTS
        pltpu.sync_copy(inst_hbm.at[pl.ds(b * _SEG, _SEG)], seg_v)
        pltpu.sync_copy(sid_hbm.at[b], sid_v)
        scls = []
        for h in range(_S // _L):
            sidx = sid_v[pl.ds(h * _L, _L)]
            scls.append(plsc.load_gather(seg_v, [sidx]))

        ones = jnp.full((_L,), 1.0, jnp.float32)
        zeros = jnp.full((_L,), 0.0, jnp.float32)

        unroll = 4

        def body(j, carry):
            i0 = j * unroll
            for u in range(unroll):
                i = i0 + u
                idxv = jnp.full((_L,), base + i, dtype=jnp.int32)
                pcls = plsc.load_gather(seg_v, [idxv])
                for h in range(_S // _L):
                    eq = pcls == scls[h]
                    out_v[i, pl.ds(h * _L, _L)] = jnp.where(eq, ones, zeros)
            return carry

        lax.fori_loop(0, _PTS // unroll, body, 0)
        pltpu.sync_copy(out_v, out_hbm.at[b, pl.ds(base, _PTS)])

    return k(instance, seed_ids)


def kernel(outputs_masks, instance, seed_ids, offset):
    nb = offset.shape[0]
    seg = outputs_masks.shape[0] // nb
    matched_outputs = outputs_masks.reshape(nb, seg, outputs_masks.shape[1])
    matched_targets = _sc_matched_targets(instance, seed_ids)
    return (matched_outputs, matched_targets)
